# baseline (speedup 1.0000x reference)
import functools

import jax
import jax.numpy as jnp
from jax import lax
from jax.experimental import pallas as pl
from jax.experimental.pallas import tpu as pltpu
from jax.experimental.pallas import tpu_sc as plsc

N_NODES = 10000
N_EDGES = 320000
D = 128
DH = 64
IDX_N = 2048

NC = 2
NS = 16
NW = NC * NS

CHUNK = 128
E_PAD = 327680
E_PER_TILE = E_PAD // NS
NCHUNK = E_PER_TILE // CHUNK

N_ACC = 10240
ROWS_PER_TILE = N_ACC // NS
PKR = 8
NRB = 4

_mesh = plsc.VectorSubcoreMesh(
    core_axis_name="c", subcore_axis_name="s", num_cores=NC, num_subcores=NS
)


def _pair_interleave(half):
    m = half.shape[0]
    return jnp.transpose(
        half.reshape(m, DH // 32, 2, 16), (0, 1, 3, 2)
    ).reshape(m, DH)


def _linear_body(x_ref, wt_ref, b_ref, o_ref):
    r = (
        jnp.dot(x_ref[...], wt_ref[...], preferred_element_type=jnp.float32)
        + b_ref[...]
    )
    o_ref[0] = _pair_interleave(r[:, :DH]).astype(jnp.bfloat16)
    o_ref[1] = _pair_interleave(r[:, DH:]).astype(jnp.bfloat16)


def _linear(x, wt, b):
    m = x.shape[0]
    bm = m // 10
    return pl.pallas_call(
        _linear_body,
        grid=(m // bm,),
        in_specs=[
            pl.BlockSpec((bm, D), lambda i: (i, 0)),
            pl.BlockSpec((D, D), lambda i: (0, 0)),
            pl.BlockSpec((1, D), lambda i: (0, 0)),
        ],
        out_specs=pl.BlockSpec((NC, bm, DH), lambda i: (0, i, 0)),
        out_shape=jax.ShapeDtypeStruct((NC, m, DH), jnp.bfloat16),
    )(x, wt, b)


def _fused_body(a_ref, wt_ref, b_ref, o_ref):
    h0 = jnp.maximum(a_ref[0], 0.0)
    h1 = jnp.maximum(a_ref[1], 0.0)
    r = (
        jnp.dot(h0, wt_ref[:DH, :], preferred_element_type=jnp.float32)
        + jnp.dot(h1, wt_ref[DH:, :], preferred_element_type=jnp.float32)
        + b_ref[...]
    )
    o_ref[0] = _pair_interleave(r[:, :DH]).astype(jnp.bfloat16)
    o_ref[1] = _pair_interleave(r[:, DH:]).astype(jnp.bfloat16)


def _relu_linear(a, wt, b):
    m = a.shape[1]
    bm = m // 10
    return pl.pallas_call(
        _fused_body,
        grid=(m // bm,),
        in_specs=[
            pl.BlockSpec((NC, bm, DH), lambda i: (0, i, 0)),
            pl.BlockSpec((D, D), lambda i: (0, 0)),
            pl.BlockSpec((1, D), lambda i: (0, 0)),
        ],
        out_specs=pl.BlockSpec((NC, bm, DH), lambda i: (0, i, 0)),
        out_shape=jax.ShapeDtypeStruct((NC, m, DH), jnp.bfloat16),
    )(a, wt, b)


def _interleave_body(a_ref, o_ref):
    o_ref[...] = jnp.concatenate([a_ref[0], a_ref[1]], axis=-1)


def _interleave(a):
    m = a.shape[1]
    bm = m // 2
    return pl.pallas_call(
        _interleave_body,
        grid=(m // bm,),
        in_specs=[pl.BlockSpec((NC, bm, DH), lambda i: (0, i, 0))],
        out_specs=pl.BlockSpec((bm, D), lambda i: (i, 0)),
        out_shape=jax.ShapeDtypeStruct((m, D), jnp.float32),
    )(a)


@functools.partial(
    pl.kernel,
    out_type=jax.ShapeDtypeStruct((NC, N_ACC, DH), jnp.float32),
    mesh=_mesh,
    scratch_types=[
        pltpu.VMEM((NCHUNK, CHUNK), jnp.float32),
        pltpu.VMEM((PKR, 2, CHUNK), jnp.int32),
        [pltpu.VMEM((CHUNK, DH), jnp.bfloat16) for _ in range(NRB)],
        [pltpu.VMEM((CHUNK, DH), jnp.float32) for _ in range(NRB)],
        pltpu.VMEM_SHARED((N_ACC, DH), jnp.float32),
        [pltpu.SemaphoreType.DMA for _ in range(NRB)],
        [pltpu.SemaphoreType.DMA for _ in range(NRB)],
        [pltpu.SemaphoreType.DMA for _ in range(PKR)],
    ],
    compiler_params=pltpu.CompilerParams(
        use_tc_tiling_on_sc=False, needs_layout_passes=False
    ),
)
def _spmm_sc(h_hbm, pk_hbm, w_hbm, out_hbm,
             w_v, pk_v, rows, sbuf, acc_sh, gsem, ssem, psem):
    c = lax.axis_index("c")
    s = lax.axis_index("s")

    pltpu.sync_copy(w_hbm.at[s], w_v)
    for m in range(5):
        pltpu.async_copy(pk_hbm.at[c, s, m], pk_v.at[m], psem[m])
    for m in range(2):
        pltpu.make_async_copy(pk_hbm.at[c, s, m], pk_v.at[m], psem[m]).wait()
        pltpu.async_copy(h_hbm.at[pk_v.at[m, 0]], rows[m], gsem[m])

    def zbody(i, _):
        sbuf[0][i // 4, pl.ds((i % 4) * 16, 16)] = jnp.zeros((16,), jnp.float32)
        return 0
    lax.fori_loop(0, CHUNK * 4, zbody, 0)
    for k in range(ROWS_PER_TILE // CHUNK):
        pltpu.sync_copy(
            sbuf[0], acc_sh.at[pl.ds(s * ROWS_PER_TILE + k * CHUNK, CHUNK)]
        )
    plsc.subcore_barrier()

    def _scale(buf, out, j):
        def group_body(g2, _):
            for gg in range(2):
                wv = w_v[j, pl.ds(g2 * 32 + gg * 16, 16)]
                for t in range(16):
                    w = wv[t]
                    row = g2 * 32 + gg * 16 + t
                    for f in range(DH // 32):
                        xs = buf[row, pl.ds(f * 32, 32)]
                        lo, hi = plsc.unpack(xs, format=plsc.PackFormat.INTERLEAVED)
                        out[row, pl.ds(f * 32, 16)] = lo * w
                        out[row, pl.ds(f * 32 + 16, 16)] = hi * w
            return 0
        lax.fori_loop(0, CHUNK // 32, group_body, 0)

    NOUT = NCHUNK // PKR

    def outer_body(j0, _):
        for b in range(PKR):
            j = j0 * PKR + b
            rb = b % NRB
            pltpu.make_async_copy(h_hbm.at[pk_v.at[b, 0]], rows[rb], gsem[rb]).wait()

            def load_pk():
                m = (b + 5) % PKR
                pltpu.async_copy(pk_hbm.at[c, s, j + 5], pk_v.at[m], psem[m])
            if b < 3:
                load_pk()
            else:
                @pl.when(j0 < NOUT - 1)
                def _():
                    load_pk()

            def launch_gather():
                m = (b + 2) % PKR
                rm = (rb + 2) % NRB
                pltpu.make_async_copy(pk_hbm.at[c, s, j + 2], pk_v.at[m], psem[m]).wait()
                pltpu.async_copy(h_hbm.at[pk_v.at[m, 0]], rows[rm], gsem[rm])
            if b < 6:
                launch_gather()
            else:
                @pl.when(j0 < NOUT - 1)
                def _():
                    launch_gather()

            def drain_prev():
                pltpu.make_async_copy(
                    sbuf[(rb + 2) % NRB], acc_sh.at[pk_v.at[b, 1]],
                    ssem[(rb + 2) % NRB],
                ).wait()
            if b < 2:
                @pl.when(j0 > 0)
                def _():
                    drain_prev()
            else:
                drain_prev()

            _scale(rows[rb], sbuf[rb], j)
            pltpu.async_copy(
                sbuf[rb], acc_sh.at[pk_v.at[b, 1]], ssem[rb], add=True
            )
    lax.fori_loop(0, NOUT, outer_body, None)

    pltpu.make_async_copy(
        sbuf[2], acc_sh.at[pk_v.at[PKR - 2, 1]], ssem[2]
    ).wait()
    pltpu.make_async_copy(
        sbuf[3], acc_sh.at[pk_v.at[PKR - 1, 1]], ssem[3]
    ).wait()
    plsc.subcore_barrier()
    pltpu.sync_copy(
        acc_sh.at[pl.ds(s * ROWS_PER_TILE, ROWS_PER_TILE)],
        out_hbm.at[c, pl.ds(s * ROWS_PER_TILE, ROWS_PER_TILE)],
    )


B_PER_W = IDX_N // NS


@functools.partial(
    pl.kernel,
    out_type=jax.ShapeDtypeStruct((NC, IDX_N, DH), jnp.float32),
    mesh=_mesh,
    scratch_types=[
        pltpu.VMEM((B_PER_W,), jnp.int32),
        pltpu.VMEM((B_PER_W, DH), jnp.float32),
        pltpu.SemaphoreType.DMA,
    ],
    compiler_params=pltpu.CompilerParams(use_tc_tiling_on_sc=False),
)
def _gather_rows(g_hbm, idx_hbm, out_hbm, idx_v, r0, sem):
    c = lax.axis_index("c")
    s = lax.axis_index("s")
    base = pl.multiple_of(s * B_PER_W, B_PER_W)
    pltpu.sync_copy(idx_hbm.at[pl.ds(base, B_PER_W)], idx_v)
    pltpu.async_copy(g_hbm.at[c].at[idx_v], r0, sem).wait()
    pltpu.sync_copy(r0, out_hbm.at[c, pl.ds(base, B_PER_W)])


def kernel(x, edge_index, edge_weight, idx, W1, b1, W2, b2):
    pad = E_PAD - N_EDGES
    src = jnp.pad(edge_index[1], (0, pad))
    pad_dst = N_NODES + (jnp.arange(pad, dtype=jnp.int32) % (N_ACC - N_NODES))
    dst = jnp.concatenate([edge_index[0], pad_dst])
    w = jnp.pad(edge_weight, (0, pad))
    src = src.reshape(NS, NCHUNK, CHUNK)
    dst = dst.reshape(NS, NCHUNK, CHUNK)
    pk = jnp.stack(
        [
            jnp.stack([src + c * N_NODES, dst], axis=2)
            for c in range(NC)
        ],
        axis=0,
    )
    pk2 = jnp.stack(
        [
            jnp.stack([src + c * N_ACC, dst], axis=2)
            for c in range(NC)
        ],
        axis=0,
    )
    w = w.reshape(NS, NCHUNK, CHUNK)

    h1 = _linear(x, W1.T, b1.reshape(1, D))
    a1 = _spmm_sc(h1.reshape(NC * N_NODES, DH), pk, w)
    h2 = _relu_linear(a1, W2.T, b2.reshape(1, D))
    a2 = _spmm_sc(h2.reshape(NC * N_ACC, DH), pk2, w)
    g = _gather_rows(a2, idx)
    return _interleave(g)


# speedup vs baseline: 1.0123x; 1.0123x over previous
import functools

import jax
import jax.numpy as jnp
from jax import lax
from jax.experimental import pallas as pl
from jax.experimental.pallas import tpu as pltpu
from jax.experimental.pallas import tpu_sc as plsc

N_NODES = 10000
N_EDGES = 320000
D = 128
DH = 64
IDX_N = 2048

NC = 2
NS = 16
NW = NC * NS

CHUNK = 128
E_PAD = 327680
E_PER_TILE = E_PAD // NS
NCHUNK = E_PER_TILE // CHUNK

N_ACC = 10240
ROWS_PER_TILE = N_ACC // NS
PKR = 8
NRB = 4

_mesh = plsc.VectorSubcoreMesh(
    core_axis_name="c", subcore_axis_name="s", num_cores=NC, num_subcores=NS
)


def _pair_interleave(half):
    m = half.shape[0]
    return jnp.transpose(
        half.reshape(m, DH // 32, 2, 16), (0, 1, 3, 2)
    ).reshape(m, DH)


def _linear_body(x_ref, wt_ref, b_ref, o_ref):
    r = (
        jnp.dot(x_ref[...], wt_ref[...], preferred_element_type=jnp.float32)
        + b_ref[...]
    )
    o_ref[0] = _pair_interleave(r[:, :DH]).astype(jnp.bfloat16)
    o_ref[1] = _pair_interleave(r[:, DH:]).astype(jnp.bfloat16)


def _linear(x, wt, b):
    m = x.shape[0]
    bm = m // 10
    return pl.pallas_call(
        _linear_body,
        grid=(m // bm,),
        in_specs=[
            pl.BlockSpec((bm, D), lambda i: (i, 0)),
            pl.BlockSpec((D, D), lambda i: (0, 0)),
            pl.BlockSpec((1, D), lambda i: (0, 0)),
        ],
        out_specs=pl.BlockSpec((NC, bm, DH), lambda i: (0, i, 0)),
        out_shape=jax.ShapeDtypeStruct((NC, m, DH), jnp.bfloat16),
    )(x, wt, b)


def _fused_body(a_ref, wt_ref, b_ref, o_ref):
    h0 = jnp.maximum(a_ref[0], 0.0)
    h1 = jnp.maximum(a_ref[1], 0.0)
    r = (
        jnp.dot(h0, wt_ref[:DH, :], preferred_element_type=jnp.float32)
        + jnp.dot(h1, wt_ref[DH:, :], preferred_element_type=jnp.float32)
        + b_ref[...]
    )
    o_ref[0] = _pair_interleave(r[:, :DH]).astype(jnp.bfloat16)
    o_ref[1] = _pair_interleave(r[:, DH:]).astype(jnp.bfloat16)


def _relu_linear(a, wt, b):
    m = a.shape[1]
    bm = m // 10
    return pl.pallas_call(
        _fused_body,
        grid=(m // bm,),
        in_specs=[
            pl.BlockSpec((NC, bm, DH), lambda i: (0, i, 0)),
            pl.BlockSpec((D, D), lambda i: (0, 0)),
            pl.BlockSpec((1, D), lambda i: (0, 0)),
        ],
        out_specs=pl.BlockSpec((NC, bm, DH), lambda i: (0, i, 0)),
        out_shape=jax.ShapeDtypeStruct((NC, m, DH), jnp.bfloat16),
    )(a, wt, b)


def _interleave_body(a_ref, o_ref):
    o_ref[...] = jnp.concatenate([a_ref[0], a_ref[1]], axis=-1)


def _interleave(a):
    m = a.shape[1]
    bm = m // 2
    return pl.pallas_call(
        _interleave_body,
        grid=(m // bm,),
        in_specs=[pl.BlockSpec((NC, bm, DH), lambda i: (0, i, 0))],
        out_specs=pl.BlockSpec((bm, D), lambda i: (i, 0)),
        out_shape=jax.ShapeDtypeStruct((m, D), jnp.float32),
    )(a)


@functools.partial(
    pl.kernel,
    out_type=jax.ShapeDtypeStruct((NC, N_ACC, DH), jnp.float32),
    mesh=_mesh,
    scratch_types=[
        pltpu.VMEM((NCHUNK, CHUNK), jnp.float32),
        pltpu.VMEM((PKR, 2, CHUNK), jnp.int32),
        [pltpu.VMEM((CHUNK, DH), jnp.bfloat16) for _ in range(NRB)],
        [pltpu.VMEM((CHUNK, DH), jnp.float32) for _ in range(NRB)],
        pltpu.VMEM_SHARED((N_ACC, DH), jnp.float32),
        [pltpu.SemaphoreType.DMA for _ in range(NRB)],
        [pltpu.SemaphoreType.DMA for _ in range(NRB)],
        [pltpu.SemaphoreType.DMA for _ in range(PKR)],
    ],
    compiler_params=pltpu.CompilerParams(
        use_tc_tiling_on_sc=False, needs_layout_passes=False
    ),
)
def _spmm_sc(h_hbm, pk_hbm, w_hbm, out_hbm,
             w_v, pk_v, rows, sbuf, acc_sh, gsem, ssem, psem):
    c = lax.axis_index("c")
    s = lax.axis_index("s")

    pltpu.sync_copy(w_hbm.at[s], w_v)
    for m in range(5):
        pltpu.async_copy(pk_hbm.at[c, s, m], pk_v.at[m], psem[m])
    for m in range(3):
        pltpu.make_async_copy(pk_hbm.at[c, s, m], pk_v.at[m], psem[m]).wait()
        pltpu.async_copy(h_hbm.at[pk_v.at[m, 0]], rows[m], gsem[m])

    def zbody(i, _):
        sbuf[0][i // 4, pl.ds((i % 4) * 16, 16)] = jnp.zeros((16,), jnp.float32)
        return 0
    lax.fori_loop(0, CHUNK * 4, zbody, 0)
    for k in range(ROWS_PER_TILE // CHUNK):
        pltpu.sync_copy(
            sbuf[0], acc_sh.at[pl.ds(s * ROWS_PER_TILE + k * CHUNK, CHUNK)]
        )
    plsc.subcore_barrier()

    def _scale(buf, out, j):
        def group_body(g, _):
            wv = w_v[j, pl.ds(g * 16, 16)]
            for t in range(16):
                w = wv[t]
                row = g * 16 + t
                for f in range(DH // 32):
                    xs = buf[row, pl.ds(f * 32, 32)]
                    lo, hi = plsc.unpack(xs, format=plsc.PackFormat.INTERLEAVED)
                    out[row, pl.ds(f * 32, 16)] = lo * w
                    out[row, pl.ds(f * 32 + 16, 16)] = hi * w
            return 0
        lax.fori_loop(0, CHUNK // 16, group_body, 0)

    NOUT = NCHUNK // PKR

    def outer_body(j0, _):
        for b in range(PKR):
            j = j0 * PKR + b
            rb = b % NRB
            pltpu.make_async_copy(h_hbm.at[pk_v.at[b, 0]], rows[rb], gsem[rb]).wait()

            def load_pk():
                m = (b + 5) % PKR
                pltpu.async_copy(pk_hbm.at[c, s, j + 5], pk_v.at[m], psem[m])
            if b < 3:
                load_pk()
            else:
                @pl.when(j0 < NOUT - 1)
                def _():
                    load_pk()

            def launch_gather():
                m = (b + 3) % PKR
                rm = (rb + 3) % NRB
                pltpu.make_async_copy(pk_hbm.at[c, s, j + 3], pk_v.at[m], psem[m]).wait()
                pltpu.async_copy(h_hbm.at[pk_v.at[m, 0]], rows[rm], gsem[rm])
            if b < 5:
                launch_gather()
            else:
                @pl.when(j0 < NOUT - 1)
                def _():
                    launch_gather()

            def drain_prev():
                pltpu.make_async_copy(
                    sbuf[(rb + 2) % NRB], acc_sh.at[pk_v.at[b, 1]],
                    ssem[(rb + 2) % NRB],
                ).wait()
            if b < 2:
                @pl.when(j0 > 0)
                def _():
                    drain_prev()
            else:
                drain_prev()

            _scale(rows[rb], sbuf[rb], j)
            pltpu.async_copy(
                sbuf[rb], acc_sh.at[pk_v.at[b, 1]], ssem[rb], add=True
            )
    lax.fori_loop(0, NOUT, outer_body, None)

    pltpu.make_async_copy(
        sbuf[2], acc_sh.at[pk_v.at[PKR - 2, 1]], ssem[2]
    ).wait()
    pltpu.make_async_copy(
        sbuf[3], acc_sh.at[pk_v.at[PKR - 1, 1]], ssem[3]
    ).wait()
    plsc.subcore_barrier()
    pltpu.sync_copy(
        acc_sh.at[pl.ds(s * ROWS_PER_TILE, ROWS_PER_TILE)],
        out_hbm.at[c, pl.ds(s * ROWS_PER_TILE, ROWS_PER_TILE)],
    )


B_PER_W = IDX_N // NS


@functools.partial(
    pl.kernel,
    out_type=jax.ShapeDtypeStruct((NC, IDX_N, DH), jnp.float32),
    mesh=_mesh,
    scratch_types=[
        pltpu.VMEM((B_PER_W,), jnp.int32),
        pltpu.VMEM((B_PER_W, DH), jnp.float32),
        pltpu.SemaphoreType.DMA,
    ],
    compiler_params=pltpu.CompilerParams(use_tc_tiling_on_sc=False),
)
def _gather_rows(g_hbm, idx_hbm, out_hbm, idx_v, r0, sem):
    c = lax.axis_index("c")
    s = lax.axis_index("s")
    base = pl.multiple_of(s * B_PER_W, B_PER_W)
    pltpu.sync_copy(idx_hbm.at[pl.ds(base, B_PER_W)], idx_v)
    pltpu.async_copy(g_hbm.at[c].at[idx_v], r0, sem).wait()
    pltpu.sync_copy(r0, out_hbm.at[c, pl.ds(base, B_PER_W)])


def kernel(x, edge_index, edge_weight, idx, W1, b1, W2, b2):
    pad = E_PAD - N_EDGES
    src = jnp.pad(edge_index[1], (0, pad))
    pad_dst = N_NODES + (jnp.arange(pad, dtype=jnp.int32) % (N_ACC - N_NODES))
    dst = jnp.concatenate([edge_index[0], pad_dst])
    w = jnp.pad(edge_weight, (0, pad))
    src = src.reshape(NS, NCHUNK, CHUNK)
    dst = dst.reshape(NS, NCHUNK, CHUNK)
    pk = jnp.stack(
        [
            jnp.stack([src + c * N_NODES, dst], axis=2)
            for c in range(NC)
        ],
        axis=0,
    )
    pk2 = jnp.stack(
        [
            jnp.stack([src + c * N_ACC, dst], axis=2)
            for c in range(NC)
        ],
        axis=0,
    )
    w = w.reshape(NS, NCHUNK, CHUNK)

    h1 = _linear(x, W1.T, b1.reshape(1, D))
    a1 = _spmm_sc(h1.reshape(NC * N_NODES, DH), pk, w)
    h2 = _relu_linear(a1, W2.T, b2.reshape(1, D))
    a2 = _spmm_sc(h2.reshape(NC * N_ACC, DH), pk2, w)
    g = _gather_rows(a2, idx)
    return _interleave(g)
